# Initial kernel scaffold; baseline (speedup 1.0000x reference)
import jax
import jax.numpy as jnp
from jax.experimental import pallas as pl


def kernel(pool_input, pool_output, inputs):
    raise NotImplementedError("write your pallas kernel here")


# speedup vs baseline: 50.9338x; 50.9338x over previous
import jax
import jax.numpy as jnp
from jax.experimental import pallas as pl

_B, _H, _W, _C = 1, 512, 512, 96
_Ho, _Wo = _H // 2, _W // 2
_HB = 16


def _unpool_kernel(pi_ref, inp_ref, out_ref):
    a = pi_ref[:, 0]
    b = pi_ref[:, 1]
    a0 = a[..., :_C]
    a1 = a[..., _C:]
    b0 = b[..., :_C]
    b1 = b[..., _C:]
    mx = jnp.maximum(jnp.maximum(a0, a1), jnp.maximum(b0, b1))
    m0 = a0 == mx
    m1 = a1 == mx
    m2 = b0 == mx
    f0 = m0
    f1 = m1 & ~m0
    f2 = m2 & ~(m0 | m1)
    f3 = ~(m0 | m1 | m2)
    v = inp_ref[...]
    z = jnp.zeros_like(v)
    out_ref[:, 0] = jnp.concatenate(
        [jnp.where(f0, v, z), jnp.where(f1, v, z)], axis=-1)
    out_ref[:, 1] = jnp.concatenate(
        [jnp.where(f2, v, z), jnp.where(f3, v, z)], axis=-1)


def kernel(pool_input, pool_output, inputs):
    del pool_output
    pi = pool_input.reshape(_Ho, 2, _Wo, 2 * _C)
    inp = inputs.reshape(_Ho, _Wo, _C)
    grid = (_Ho // _HB,)
    out = pl.pallas_call(
        _unpool_kernel,
        grid=grid,
        in_specs=[
            pl.BlockSpec((_HB, 2, _Wo, 2 * _C), lambda i: (i, 0, 0, 0)),
            pl.BlockSpec((_HB, _Wo, _C), lambda i: (i, 0, 0)),
        ],
        out_specs=pl.BlockSpec((_HB, 2, _Wo, 2 * _C), lambda i: (i, 0, 0, 0)),
        out_shape=jax.ShapeDtypeStruct((_Ho, 2, _Wo, 2 * _C), inputs.dtype),
    )(pi, inp)
    return out.reshape(_B, _H, _W, _C)


# speedup vs baseline: 82.1158x; 1.6122x over previous
import jax
import jax.numpy as jnp
from jax.experimental import pallas as pl

_B, _H, _W, _C = 1, 512, 512, 96
_Ho, _Wo = _H // 2, _W // 2
_HB = 16


def _unpool_kernel(pi_ref, inp_ref, out_ref):
    x = pi_ref[...].reshape(_HB, 2, _W, _C)
    a = x[:, 0]
    b = x[:, 1]
    col = jax.lax.broadcasted_iota(jnp.int32, (_HB, _W, _C), 1)
    even = (col % 2) == 0
    al = jnp.roll(a, -1, axis=1)
    ar = jnp.roll(a, 1, axis=1)
    bl = jnp.roll(b, -1, axis=1)
    br = jnp.roll(b, 1, axis=1)
    a_o = jnp.where(even, al, ar)
    b_o = jnp.where(even, bl, br)
    mx = jnp.maximum(jnp.maximum(a, a_o), jnp.maximum(b, b_o))
    m_a = a == mx
    m_b = b == mx
    m_a_r = ar == mx
    m_a_l = al == mx
    m_b_r = br == mx
    f_a = m_a & (even | ~m_a_r)
    any_a = m_a | (even & m_a_l) | (~even & m_a_r)
    f_b = m_b & ~any_a & (even | ~m_b_r)
    v = jnp.repeat(inp_ref[...], 2, axis=1)
    z = jnp.zeros_like(v)
    oa = jnp.where(f_a, v, z)
    ob = jnp.where(f_b, v, z)
    out_ref[...] = jnp.stack([oa, ob], axis=1).reshape(2 * _HB, _W, _C)


def kernel(pool_input, pool_output, inputs):
    del pool_output
    pi = pool_input.reshape(_H, _W, _C)
    inp = inputs.reshape(_Ho, _Wo, _C)
    out = pl.pallas_call(
        _unpool_kernel,
        grid=(_Ho // _HB,),
        in_specs=[
            pl.BlockSpec((2 * _HB, _W, _C), lambda i: (i, 0, 0)),
            pl.BlockSpec((_HB, _Wo, _C), lambda i: (i, 0, 0)),
        ],
        out_specs=pl.BlockSpec((2 * _HB, _W, _C), lambda i: (i, 0, 0)),
        out_shape=jax.ShapeDtypeStruct((_H, _W, _C), inputs.dtype),
    )(pi, inp)
    return out.reshape(_B, _H, _W, _C)
